# baseline (speedup 1.0000x reference)
import functools

import jax
import jax.numpy as jnp
from jax import lax
from jax.experimental import pallas as pl
from jax.experimental.pallas import tpu as pltpu
from jax.experimental.pallas import tpu_sc as plsc

NC = 2
NS = 16
NW = NC * NS
L = 16

D = 128
DP = D // 2
NBUF = 4
TSTRIDE = L + 1

C1 = 64
CHUNKS1 = 208
BPW1 = C1 * CHUNKS1
E1 = BPW1 * NW

C2 = 128
CHUNKS2 = 20
BPW2 = C2 * CHUNKS2
E2 = BPW2 * NW

E_PAD = E1 + E2


def _dot_f32(u_r, i_r, e):
    acc = u_r[e, pl.ds(0, L)] * i_r[e, pl.ds(0, L)]
    for t in range(1, D // L):
        acc = acc + u_r[e, pl.ds(t * L, L)] * i_r[e, pl.ds(t * L, L)]
    return acc


def _dot_packed(u_r, i_r, e):
    acc = jnp.zeros((L,), jnp.float32)
    for t in range(DP // L):
        uw = u_r[e, pl.ds(t * L, L)]
        iw = i_r[e, pl.ds(t * L, L)]
        ua = plsc.bitcast(uw << 16, jnp.float32)
        ub = plsc.bitcast(uw, jnp.float32)
        ia = plsc.bitcast(iw << 16, jnp.float32)
        ib = plsc.bitcast(iw, jnp.float32)
        acc = acc + (ua * ia + ub * ib)
    return acc


def _make_body(c, chunks, bpw, e_off, dot_fn):
    def body(user_hbm, item_hbm, src_hbm, dst_hbm, out_hbm,
             src_v, dst_v, u_bufs, i_bufs, sc_all, tmp, sem_u, sem_i):
        wid = lax.axis_index("s") * NC + lax.axis_index("c")
        base = e_off + wid * bpw
        pltpu.sync_copy(src_hbm.at[pl.ds(base, bpw)], src_v)
        pltpu.sync_copy(dst_hbm.at[pl.ds(base, bpw)], dst_v)

        def issue(k, b):
            pltpu.async_copy(user_hbm.at[src_v.at[pl.ds(k * c, c)]],
                             u_bufs[b], sem_u[b])
            pltpu.async_copy(item_hbm.at[dst_v.at[pl.ds(k * c, c)]],
                             i_bufs[b], sem_i[b])

        lane = lax.iota(jnp.int32, L)
        col0 = lane * TSTRIDE

        def compute(k, b):
            u_r = u_bufs[b]
            i_r = i_bufs[b]
            pltpu.make_async_copy(user_hbm.at[src_v.at[pl.ds(k * c, c)]],
                                  u_r, sem_u[b]).wait()
            pltpu.make_async_copy(item_hbm.at[dst_v.at[pl.ds(k * c, c)]],
                                  i_r, sem_i[b]).wait()

            def group_body(g, _):
                e0 = g * L
                for j in range(L):
                    tmp[pl.ds(j * TSTRIDE, L)] = dot_fn(u_r, i_r, e0 + j)
                scores = plsc.load_gather(tmp, [col0])
                for k2 in range(1, L):
                    scores = scores + plsc.load_gather(tmp, [col0 + k2])
                sc_all[pl.ds(k * c + e0, L)] = scores
                return ()

            lax.fori_loop(0, c // L, group_body, (), unroll=False)

        for b in range(NBUF):
            issue(b, b)

        def loop(q, _):
            k0 = q * NBUF
            for b in range(NBUF):
                compute(k0 + b, b)

                @pl.when(k0 + b + NBUF < chunks)
                def _():
                    issue(k0 + b + NBUF, b)
            return ()

        lax.fori_loop(0, chunks // NBUF, loop, (), unroll=False)
        pltpu.sync_copy(sc_all, out_hbm.at[pl.ds(wid * bpw, bpw)])

    return body


def _phase_call(body, c, bpw, total, buf_shape, buf_dtype, tc_tiling):
    mesh = plsc.VectorSubcoreMesh(core_axis_name="c", subcore_axis_name="s")
    return pl.kernel(
        body,
        out_type=jax.ShapeDtypeStruct((total,), jnp.float32),
        mesh=mesh,
        compiler_params=pltpu.CompilerParams(
            needs_layout_passes=False, use_tc_tiling_on_sc=tc_tiling),
        scratch_types=[
            pltpu.VMEM((bpw,), jnp.int32),
            pltpu.VMEM((bpw,), jnp.int32),
            [pltpu.VMEM(buf_shape, buf_dtype) for _ in range(NBUF)],
            [pltpu.VMEM(buf_shape, buf_dtype) for _ in range(NBUF)],
            pltpu.VMEM((bpw,), jnp.float32),
            pltpu.VMEM((L * TSTRIDE,), jnp.float32),
            [pltpu.SemaphoreType.DMA for _ in range(NBUF)],
            [pltpu.SemaphoreType.DMA for _ in range(NBUF)],
        ],
    )


def _pack_bf16_words(table):
    u = jax.lax.bitcast_convert_type(table, jnp.uint32)
    r = u + jnp.uint32(0x7FFF) + ((u >> 16) & jnp.uint32(1))
    lo = r[:, :DP] >> 16
    hi = r[:, DP:] & jnp.uint32(0xFFFF0000)
    return jax.lax.bitcast_convert_type(lo | hi, jnp.int32)


@jax.jit
def _run(user_table, item_table, edge_label_index):
    e = edge_label_index.shape[1]
    pad = E_PAD - e
    pad_idx = jnp.arange(pad, dtype=jnp.int32) % user_table.shape[0]
    src = jnp.concatenate([edge_label_index[0], pad_idx])
    dst = jnp.concatenate([edge_label_index[1], pad_idx])

    s1 = _phase_call(_make_body(C1, CHUNKS1, BPW1, 0, _dot_f32),
                     C1, BPW1, E1, (C1, D), jnp.float32, None)(
                         user_table, item_table, src, dst)
    s2 = _phase_call(_make_body(C2, CHUNKS2, BPW2, E1, _dot_packed),
                     C2, BPW2, E2, (C2, DP), jnp.int32, False)(
                         _pack_bf16_words(user_table),
                         _pack_bf16_words(item_table), src, dst)
    return jnp.concatenate([s1, s2])[:e]


def kernel(user_table, item_table, edge_label_index):
    return _run(user_table, item_table, edge_label_index)


# speedup vs baseline: 1.1569x; 1.1569x over previous
import functools

import jax
import jax.numpy as jnp
from jax import lax
from jax.experimental import pallas as pl
from jax.experimental.pallas import tpu as pltpu
from jax.experimental.pallas import tpu_sc as plsc

NC = 2
NS = 16
NW = NC * NS
L = 16

D = 128
DP = D // 2
NBUF = 4
TSTRIDE = L + 1

C1 = 64
CHUNKS1 = 208
BPW1 = C1 * CHUNKS1
E1 = BPW1 * NW

C2 = 128
CHUNKS2 = 20
BPW2 = C2 * CHUNKS2
E2 = BPW2 * NW

E_PAD = E1 + E2


def _dot_f32(u_r, i_r, e):
    acc = u_r[e, pl.ds(0, L)] * i_r[e, pl.ds(0, L)]
    for t in range(1, D // L):
        acc = acc + u_r[e, pl.ds(t * L, L)] * i_r[e, pl.ds(t * L, L)]
    return acc


def _dot_packed(u_r, i_r, e):
    acc = jnp.zeros((L,), jnp.float32)
    for t in range(DP // L):
        uw = u_r[e, pl.ds(t * L, L)]
        iw = i_r[e, pl.ds(t * L, L)]
        ua = plsc.bitcast(uw << 16, jnp.float32)
        ub = plsc.bitcast(uw, jnp.float32)
        ia = plsc.bitcast(iw << 16, jnp.float32)
        ib = plsc.bitcast(iw, jnp.float32)
        acc = acc + (ua * ia + ub * ib)
    return acc


def _make_body(c, chunks, bpw, e_off, dot_fn):
    def body(user_hbm, item_hbm, src_hbm, dst_hbm, out_hbm,
             src_v, dst_v, u_bufs, i_bufs, sc_all, tmp, sem_u, sem_i):
        wid = lax.axis_index("s") * NC + lax.axis_index("c")
        base = e_off + wid * bpw
        pltpu.sync_copy(src_hbm.at[pl.ds(base, bpw)], src_v)
        pltpu.sync_copy(dst_hbm.at[pl.ds(base, bpw)], dst_v)

        def issue(k, b):
            pltpu.async_copy(user_hbm.at[src_v.at[pl.ds(k * c, c)]],
                             u_bufs[b], sem_u[b])
            pltpu.async_copy(item_hbm.at[dst_v.at[pl.ds(k * c, c)]],
                             i_bufs[b], sem_i[b])

        lane = lax.iota(jnp.int32, L)
        col0 = lane * TSTRIDE

        def compute(k, b):
            u_r = u_bufs[b]
            i_r = i_bufs[b]
            pltpu.make_async_copy(user_hbm.at[src_v.at[pl.ds(k * c, c)]],
                                  u_r, sem_u[b]).wait()
            pltpu.make_async_copy(item_hbm.at[dst_v.at[pl.ds(k * c, c)]],
                                  i_r, sem_i[b]).wait()

            def group_body(g, _):
                e0 = g * L
                for j in range(L):
                    tmp[pl.ds(j * TSTRIDE, L)] = dot_fn(u_r, i_r, e0 + j)
                scores = plsc.load_gather(tmp, [col0])
                for k2 in range(1, L):
                    scores = scores + plsc.load_gather(tmp, [col0 + k2])
                sc_all[pl.ds(k * c + e0, L)] = scores
                return ()

            lax.fori_loop(0, c // L, group_body, (), unroll=False)

        for b in range(NBUF):
            issue(b, b)

        def loop(q, _):
            k0 = q * NBUF
            for b in range(NBUF):
                compute(k0 + b, b)

                @pl.when(k0 + b + NBUF < chunks)
                def _():
                    issue(k0 + b + NBUF, b)
            return ()

        lax.fori_loop(0, chunks // NBUF, loop, (), unroll=False)
        pltpu.sync_copy(sc_all, out_hbm.at[pl.ds(wid * bpw, bpw)])

    return body


def _phase_call(body, c, bpw, total, buf_shape, buf_dtype, tc_tiling):
    mesh = plsc.VectorSubcoreMesh(core_axis_name="c", subcore_axis_name="s")
    return pl.kernel(
        body,
        out_type=jax.ShapeDtypeStruct((total,), jnp.float32),
        mesh=mesh,
        compiler_params=pltpu.CompilerParams(
            needs_layout_passes=False, use_tc_tiling_on_sc=tc_tiling),
        scratch_types=[
            pltpu.VMEM((bpw,), jnp.int32),
            pltpu.VMEM((bpw,), jnp.int32),
            [pltpu.VMEM(buf_shape, buf_dtype) for _ in range(NBUF)],
            [pltpu.VMEM(buf_shape, buf_dtype) for _ in range(NBUF)],
            pltpu.VMEM((bpw,), jnp.float32),
            pltpu.VMEM((L * TSTRIDE,), jnp.float32),
            [pltpu.SemaphoreType.DMA for _ in range(NBUF)],
            [pltpu.SemaphoreType.DMA for _ in range(NBUF)],
        ],
    )


def _pack_bf16_words(table):
    u = jax.lax.bitcast_convert_type(table, jnp.uint32)
    r = u + jnp.uint32(0x7FFF) + ((u >> 16) & jnp.uint32(1))
    lo = r[:, :DP] >> 16
    hi = r[:, DP:] & jnp.uint32(0xFFFF0000)
    return jax.lax.bitcast_convert_type(lo | hi, jnp.int32)


@jax.jit
def _run(user_table, item_table, edge_label_index):
    e = edge_label_index.shape[1]
    pad = E_PAD - e
    pad_idx = jnp.arange(pad, dtype=jnp.int32) % user_table.shape[0]
    src = jnp.concatenate([edge_label_index[0], pad_idx])
    dst = jnp.concatenate([edge_label_index[1], pad_idx])

    s1 = _phase_call(_make_body(C1, CHUNKS1, BPW1, 0, _dot_f32),
                     C1, BPW1, E1, (C1, D), jnp.float32, None)(
                         user_table, item_table, src, dst)
    up = _pack_bf16_words(user_table)
    ip = _pack_bf16_words(item_table)
    s1, up, ip = jax.lax.optimization_barrier((s1, up, ip))
    s2 = _phase_call(_make_body(C2, CHUNKS2, BPW2, E1, _dot_packed),
                     C2, BPW2, E2, (C2, DP), jnp.int32, False)(
                         up, ip, src, dst)
    return jnp.concatenate([s1, s2])[:e]


def kernel(user_table, item_table, edge_label_index):
    return _run(user_table, item_table, edge_label_index)


# speedup vs baseline: 1.7046x; 1.4734x over previous
import functools

import jax
import jax.numpy as jnp
from jax import lax
from jax.experimental import pallas as pl
from jax.experimental.pallas import tpu as pltpu
from jax.experimental.pallas import tpu_sc as plsc

NC = 2
NS = 16
NW = NC * NS
L = 16

D = 128
C = 64
NBUF = 4
CHUNKS = 248
BPW = C * CHUNKS
E_PAD = BPW * NW
TSTRIDE = L + 1


def _sc_body(user_hbm, item_hbm, src_hbm, dst_hbm, out_hbm,
             src_v, dst_v, u_bufs, i_bufs, sc_all, tmp, sem_u, sem_i):
    wid = lax.axis_index("s") * NC + lax.axis_index("c")
    base = wid * BPW
    pltpu.sync_copy(src_hbm.at[pl.ds(base, BPW)], src_v)
    pltpu.sync_copy(dst_hbm.at[pl.ds(base, BPW)], dst_v)

    def issue(k, b):
        pltpu.async_copy(user_hbm.at[src_v.at[pl.ds(k * C, C)]],
                         u_bufs[b], sem_u[b])
        pltpu.async_copy(item_hbm.at[dst_v.at[pl.ds(k * C, C)]],
                         i_bufs[b], sem_i[b])

    lane = lax.iota(jnp.int32, L)
    col0 = lane * TSTRIDE

    def compute(k, b):
        u_r = u_bufs[b]
        i_r = i_bufs[b]
        pltpu.make_async_copy(user_hbm.at[src_v.at[pl.ds(k * C, C)]],
                              u_r, sem_u[b]).wait()
        pltpu.make_async_copy(item_hbm.at[dst_v.at[pl.ds(k * C, C)]],
                              i_r, sem_i[b]).wait()

        def group_body(g, _):
            e0 = g * L
            for j in range(L):
                acc = u_r[e0 + j, pl.ds(0, L)] * i_r[e0 + j, pl.ds(0, L)]
                for t in range(1, D // L):
                    acc = acc + (u_r[e0 + j, pl.ds(t * L, L)]
                                 * i_r[e0 + j, pl.ds(t * L, L)])
                tmp[pl.ds(j * TSTRIDE, L)] = acc
            scores = plsc.load_gather(tmp, [col0])
            for k2 in range(1, L):
                scores = scores + plsc.load_gather(tmp, [col0 + k2])
            sc_all[pl.ds(k * C + e0, L)] = scores
            return ()

        lax.fori_loop(0, C // L, group_body, (), unroll=False)

    for b in range(NBUF):
        issue(b, b)

    def body(q, _):
        k0 = q * NBUF
        for b in range(NBUF):
            compute(k0 + b, b)

            @pl.when(k0 + b + NBUF < CHUNKS)
            def _():
                issue(k0 + b + NBUF, b)
        return ()

    lax.fori_loop(0, CHUNKS // NBUF, body, (), unroll=False)
    pltpu.sync_copy(sc_all, out_hbm.at[pl.ds(base, BPW)])


def _sc_scores(user_table, item_table, src, dst):
    mesh = plsc.VectorSubcoreMesh(core_axis_name="c", subcore_axis_name="s")
    return pl.kernel(
        _sc_body,
        out_type=jax.ShapeDtypeStruct((E_PAD,), jnp.float32),
        mesh=mesh,
        compiler_params=pltpu.CompilerParams(needs_layout_passes=False),
        scratch_types=[
            pltpu.VMEM((BPW,), jnp.int32),
            pltpu.VMEM((BPW,), jnp.int32),
            [pltpu.VMEM((C, D), jnp.float32) for _ in range(NBUF)],
            [pltpu.VMEM((C, D), jnp.float32) for _ in range(NBUF)],
            pltpu.VMEM((BPW,), jnp.float32),
            pltpu.VMEM((L * TSTRIDE,), jnp.float32),
            [pltpu.SemaphoreType.DMA for _ in range(NBUF)],
            [pltpu.SemaphoreType.DMA for _ in range(NBUF)],
        ],
    )(user_table, item_table, src, dst)


@jax.jit
def _run(user_table, item_table, edge_label_index):
    e = edge_label_index.shape[1]
    pad = E_PAD - e
    pad_idx = jnp.arange(pad, dtype=jnp.int32) % user_table.shape[0]
    src = jnp.concatenate([edge_label_index[0], pad_idx])
    dst = jnp.concatenate([edge_label_index[1], pad_idx])
    return _sc_scores(user_table, item_table, src, dst)[:e]


def kernel(user_table, item_table, edge_label_index):
    return _run(user_table, item_table, edge_label_index)
